# baseline (speedup 1.0000x reference)
import functools

import jax
import jax.numpy as jnp
import numpy as np
from jax import lax
from jax.experimental import pallas as pl
from jax.experimental.pallas import tpu as pltpu
from jax.experimental.pallas import tpu_sc as plsc

_SC_CORES = 2
_SC_SUBCORES = 16


def _segments(n, k):
    stride = n // k
    idx = np.linspace(0, n - 1, k).astype(np.int64)
    d = idx - stride * np.arange(k)
    segs = []
    s0 = 0
    for i in range(1, k + 1):
        if i == k or d[i] != d[s0]:
            segs.append((int(s0), int(i), int(d[s0])))
            s0 = i
    return stride, segs


def _gather_kernel(segs, kk, vra, mra, vrb, mrb, x):
    g = pl.program_id(1)
    for s0, s1, d in segs:
        @pl.when(g == d)
        def _(s0=s0, s1=s1):
            vd = vra.shape[2]
            md = mra.shape[2]
            x[0, s0:s1, 0:vd] = vra[0, s0:s1, :].astype(jnp.bfloat16)
            x[0, s0:s1, vd:vd + md] = mra[0, s0:s1, :].astype(jnp.bfloat16)
            x[0, kk + s0:kk + s1, 0:vd] = vrb[0, s0:s1, :].astype(jnp.bfloat16)
            x[0, kk + s0:kk + s1, vd:vd + md] = (
                mrb[0, s0:s1, :].astype(jnp.bfloat16))


def _qkv_kernel(vd, x, wq, wk, wv, q, k, v):
    xx = x[0]
    q[0] = jnp.dot(xx, wq[...],
                   preferred_element_type=jnp.float32).astype(jnp.bfloat16)
    k[0] = jnp.dot(xx, wk[...],
                   preferred_element_type=jnp.float32).astype(jnp.bfloat16)
    v[0] = jnp.dot(xx[:, :vd], wv[...],
                   preferred_element_type=jnp.float32).astype(jnp.bfloat16)


def _attn_kernel(scale, dh, nh, q, k, v, wo, u, acc):
    h = pl.program_id(1)
    qq, kk, vv, woo = q[0], k[0], v[0], wo[...]
    contrib = None
    for j in range(qq.shape[-1] // dh):
        qh = qq[:, j * dh:(j + 1) * dh]
        kh = kk[:, j * dh:(j + 1) * dh]
        vh = vv[:, j * dh:(j + 1) * dh]
        s = jax.lax.dot_general(qh, kh, (((1,), (1,)), ((), ())),
                                preferred_element_type=jnp.float32) * scale
        m = jnp.max(s, axis=-1, keepdims=True)
        p = jnp.exp(s - m)
        l = jnp.sum(p, axis=-1, keepdims=True)
        o = jnp.dot(p.astype(jnp.bfloat16), vh,
                    preferred_element_type=jnp.float32) / l
        c = jnp.dot(o.astype(jnp.bfloat16),
                    woo[j * dh:(j + 1) * dh, :],
                    preferred_element_type=jnp.float32)
        contrib = c if contrib is None else contrib + c

    @pl.when(h == 0)
    def _():
        acc[...] = contrib

    @pl.when(h > 0)
    def _():
        acc[...] += contrib

    @pl.when(h == nh - 1)
    def _():
        u[0] = acc[...].astype(jnp.bfloat16)


def _sc_copy_body(nw, per_w, sa, sb, oa, ob):
    w = lax.axis_index("s") * _SC_CORES + lax.axis_index("c")
    base = w * per_w
    pltpu.sync_copy(sa.at[pl.ds(base, per_w)], oa.at[pl.ds(base, per_w)])
    pltpu.sync_copy(sb.at[pl.ds(base, per_w)], ob.at[pl.ds(base, per_w)])


def _add_kernel(tab, tabr, tabg, o0a, o0b, uu, oa, ob):
    t = pl.program_id(1)
    oa[0] = o0a[0]
    ob[0] = o0b[0]
    for tt, (_, _, lo, hi) in enumerate(tab):
        @pl.when(t == tt)
        def _(lo=lo, hi=hi):
            oa[0, lo:hi, :] += uu[0, 0, lo:hi, :].astype(jnp.float32)
            ob[0, lo:hi, :] += uu[0, 1, lo:hi, :].astype(jnp.float32)


def kernel(values_a, metadata_a, values_b, metadata_b, Wq, Wk, Wv, Wo):
    B, N, VD = values_a.shape
    MD = metadata_a.shape[2]
    ID = Wq.shape[1]
    K = ID
    H = 16
    dh = ID // H

    stride, segs = _segments(N, K)
    R = N // stride

    vra = values_a.reshape(B, R, stride * VD)
    vrb = values_b.reshape(B, R, stride * VD)
    mra = metadata_a.reshape(B, R, stride * MD)
    mrb = metadata_b.reshape(B, R, stride * MD)

    NW = _SC_CORES * _SC_SUBCORES
    tot = B * N * VD
    per_w = tot // NW
    sc_copy = functools.partial(
        pl.kernel,
        out_type=[jax.ShapeDtypeStruct((tot,), jnp.float32)] * 2,
        mesh=plsc.VectorSubcoreMesh(
            core_axis_name="c", subcore_axis_name="s",
            num_cores=_SC_CORES, num_subcores=_SC_SUBCORES),
    )(functools.partial(_sc_copy_body, NW, per_w))
    o0a, o0b = sc_copy(values_a.reshape(tot), values_b.reshape(tot))
    o0a = o0a.reshape(B, R, stride * VD)
    o0b = o0b.reshape(B, R, stride * VD)

    T = 2 * K
    x = pl.pallas_call(
        functools.partial(_gather_kernel, segs, K),
        grid=(B, stride),
        in_specs=[
            pl.BlockSpec((1, R, VD), lambda b, g: (b, 0, g)),
            pl.BlockSpec((1, R, MD), lambda b, g: (b, 0, g)),
            pl.BlockSpec((1, R, VD), lambda b, g: (b, 0, g)),
            pl.BlockSpec((1, R, MD), lambda b, g: (b, 0, g)),
        ],
        out_specs=pl.BlockSpec((1, T, VD + MD), lambda b, g: (b, 0, 0)),
        out_shape=jax.ShapeDtypeStruct((B, T, VD + MD), jnp.bfloat16),
    )(vra, mra, vrb, mrb)

    RB = 2
    q, k, v = pl.pallas_call(
        functools.partial(_qkv_kernel, VD),
        grid=(B, RB),
        in_specs=[
            pl.BlockSpec((1, T // RB, VD + MD), lambda b, r: (b, r, 0)),
            pl.BlockSpec((VD + MD, ID), lambda b, r: (0, 0)),
            pl.BlockSpec((VD + MD, ID), lambda b, r: (0, 0)),
            pl.BlockSpec((VD, ID), lambda b, r: (0, 0)),
        ],
        out_specs=[pl.BlockSpec((1, T // RB, ID), lambda b, r: (b, r, 0))] * 3,
        out_shape=[jax.ShapeDtypeStruct((B, T, ID), jnp.bfloat16)] * 3,
    )(x, Wq.astype(jnp.bfloat16), Wk.astype(jnp.bfloat16),
      Wv.astype(jnp.bfloat16))

    HPB = 2
    NH = H // HPB
    hspec = pl.BlockSpec((1, T, HPB * dh), lambda b, h: (b, 0, h))
    u = pl.pallas_call(
        functools.partial(_attn_kernel, 1.0 / np.sqrt(dh), dh, NH),
        grid=(B, NH),
        in_specs=[
            hspec, hspec, hspec,
            pl.BlockSpec((HPB * dh, VD), lambda b, h: (h, 0)),
        ],
        out_specs=pl.BlockSpec((1, T, VD), lambda b, h: (b, 0, 0)),
        out_shape=jax.ShapeDtypeStruct((B, T, VD), jnp.bfloat16),
        scratch_shapes=[pltpu.VMEM((T, VD), jnp.float32)],
    )(q, k, v, Wo.astype(jnp.bfloat16))

    ur = u.reshape(B, 2, K, VD)

    RBROWS = 128
    tab = []
    for s0, s1, d in segs:
        for rb in range(s0 // RBROWS, (s1 - 1) // RBROWS + 1):
            lo = max(s0, rb * RBROWS) - rb * RBROWS
            hi = min(s1, (rb + 1) * RBROWS) - rb * RBROWS
            tab.append((rb, d, lo, hi))
    tab_r = jnp.asarray([e[0] for e in tab], jnp.int32)
    tab_g = jnp.asarray([e[1] for e in tab], jnp.int32)

    ospec = pl.BlockSpec((1, RBROWS, VD),
                         lambda b, t, tr, tg: (b, tr[t], tg[t]))
    oa, ob = pl.pallas_call(
        functools.partial(_add_kernel, tab),
        grid_spec=pltpu.PrefetchScalarGridSpec(
            num_scalar_prefetch=2,
            grid=(B, len(tab)),
            in_specs=[
                ospec, ospec,
                pl.BlockSpec((1, 2, RBROWS, VD),
                             lambda b, t, tr, tg: (b, 0, tr[t], 0)),
            ],
            out_specs=[ospec, ospec],
        ),
        out_shape=[jax.ShapeDtypeStruct((B, R, stride * VD), jnp.float32)] * 2,
        input_output_aliases={2: 0, 3: 1},
    )(tab_r, tab_g, o0a, o0b, ur)
    return oa.reshape(B, N, VD), ob.reshape(B, N, VD)


# speedup vs baseline: 3.0749x; 3.0749x over previous
import functools

import jax
import jax.numpy as jnp
import numpy as np
from jax import lax
from jax.experimental import pallas as pl
from jax.experimental.pallas import tpu as pltpu
from jax.experimental.pallas import tpu_sc as plsc

_SC_CORES = 2
_SC_SUBCORES = 16


def _segments(n, k):
    stride = n // k
    idx = np.linspace(0, n - 1, k).astype(np.int64)
    d = idx - stride * np.arange(k)
    segs = []
    s0 = 0
    for i in range(1, k + 1):
        if i == k or d[i] != d[s0]:
            segs.append((int(s0), int(i), int(d[s0])))
            s0 = i
    return stride, segs


def _gather_kernel(segs, kk, vra, mra, vrb, mrb, x):
    g = pl.program_id(1)
    for s0, s1, d in segs:
        @pl.when(g == d)
        def _(s0=s0, s1=s1):
            vd = vra.shape[2]
            md = mra.shape[2]
            x[0, s0:s1, 0:vd] = vra[0, s0:s1, :].astype(jnp.bfloat16)
            x[0, s0:s1, vd:vd + md] = mra[0, s0:s1, :].astype(jnp.bfloat16)
            x[0, kk + s0:kk + s1, 0:vd] = vrb[0, s0:s1, :].astype(jnp.bfloat16)
            x[0, kk + s0:kk + s1, vd:vd + md] = (
                mrb[0, s0:s1, :].astype(jnp.bfloat16))


def _qkv_kernel(vd, x, wq, wk, wv, q, k, v):
    xx = x[0]
    q[0] = jnp.dot(xx, wq[...],
                   preferred_element_type=jnp.float32).astype(jnp.bfloat16)
    k[0] = jnp.dot(xx, wk[...],
                   preferred_element_type=jnp.float32).astype(jnp.bfloat16)
    v[0] = jnp.dot(xx[:, :vd], wv[...],
                   preferred_element_type=jnp.float32).astype(jnp.bfloat16)


def _attn_kernel(scale, dh, nh, q, k, v, wo, u, acc):
    h = pl.program_id(1)
    qq, kk, vv, woo = q[0], k[0], v[0], wo[...]
    contrib = None
    for j in range(qq.shape[-1] // dh):
        qh = qq[:, j * dh:(j + 1) * dh]
        kh = kk[:, j * dh:(j + 1) * dh]
        vh = vv[:, j * dh:(j + 1) * dh]
        s = jax.lax.dot_general(qh, kh, (((1,), (1,)), ((), ())),
                                preferred_element_type=jnp.float32) * scale
        m = jnp.max(s, axis=-1, keepdims=True)
        p = jnp.exp(s - m)
        l = jnp.sum(p, axis=-1, keepdims=True)
        o = jnp.dot(p.astype(jnp.bfloat16), vh,
                    preferred_element_type=jnp.float32) / l
        c = jnp.dot(o.astype(jnp.bfloat16),
                    woo[j * dh:(j + 1) * dh, :],
                    preferred_element_type=jnp.float32)
        contrib = c if contrib is None else contrib + c

    @pl.when(h == 0)
    def _():
        acc[...] = contrib

    @pl.when(h > 0)
    def _():
        acc[...] += contrib

    @pl.when(h == nh - 1)
    def _():
        u[0] = acc[...].astype(jnp.bfloat16)


def _sc_copy_body(nw, per_w, sa, sb, oa, ob, buf0, buf1, s0, s1, s2, s3):
    w = lax.axis_index("s") * _SC_CORES + lax.axis_index("c")
    base = w * per_w
    bufs, isems, osems = (buf0, buf1), (s0, s1), (s2, s3)
    ch = buf0.shape[0]
    outcp = [None, None]
    for j, (src, dst) in enumerate(((sa, oa), (sb, ob))):
        for i in range(per_w // ch):
            b = i % 2
            if outcp[b] is not None:
                outcp[b].wait()
            pltpu.async_copy(src.at[pl.ds(base + i * ch, ch)], bufs[b],
                             isems[b]).wait()
            outcp[b] = pltpu.async_copy(bufs[b],
                                        dst.at[pl.ds(base + i * ch, ch)],
                                        osems[b])
    for c in outcp:
        if c is not None:
            c.wait()


def _add_kernel(tab, tabr, tabg, o0a, o0b, uu, oa, ob):
    t = pl.program_id(1)
    oa[0] = o0a[0]
    ob[0] = o0b[0]
    for tt, (_, _, lo, hi) in enumerate(tab):
        @pl.when(t == tt)
        def _(lo=lo, hi=hi):
            oa[0, lo:hi, :] += uu[0, 0, lo:hi, :].astype(jnp.float32)
            ob[0, lo:hi, :] += uu[0, 1, lo:hi, :].astype(jnp.float32)


def kernel(values_a, metadata_a, values_b, metadata_b, Wq, Wk, Wv, Wo):
    B, N, VD = values_a.shape
    MD = metadata_a.shape[2]
    ID = Wq.shape[1]
    K = ID
    H = 16
    dh = ID // H

    stride, segs = _segments(N, K)
    R = N // stride

    vra = values_a.reshape(B, R, stride * VD)
    vrb = values_b.reshape(B, R, stride * VD)
    mra = metadata_a.reshape(B, R, stride * MD)
    mrb = metadata_b.reshape(B, R, stride * MD)

    NW = _SC_CORES * _SC_SUBCORES
    tot = B * N * VD
    per_w = tot // NW
    CH = 32768
    sc_copy = functools.partial(
        pl.kernel,
        out_type=[jax.ShapeDtypeStruct((tot,), jnp.float32)] * 2,
        mesh=plsc.VectorSubcoreMesh(
            core_axis_name="c", subcore_axis_name="s",
            num_cores=_SC_CORES, num_subcores=_SC_SUBCORES),
        scratch_types=[
            pltpu.VMEM((CH,), jnp.float32),
            pltpu.VMEM((CH,), jnp.float32),
            pltpu.SemaphoreType.DMA,
            pltpu.SemaphoreType.DMA,
            pltpu.SemaphoreType.DMA,
            pltpu.SemaphoreType.DMA,
        ],
    )(functools.partial(_sc_copy_body, NW, per_w))
    o0a, o0b = sc_copy(values_a.reshape(tot), values_b.reshape(tot))
    o0a = o0a.reshape(B, R, stride * VD)
    o0b = o0b.reshape(B, R, stride * VD)

    T = 2 * K
    x = pl.pallas_call(
        functools.partial(_gather_kernel, segs, K),
        grid=(B, stride),
        in_specs=[
            pl.BlockSpec((1, R, VD), lambda b, g: (b, 0, g)),
            pl.BlockSpec((1, R, MD), lambda b, g: (b, 0, g)),
            pl.BlockSpec((1, R, VD), lambda b, g: (b, 0, g)),
            pl.BlockSpec((1, R, MD), lambda b, g: (b, 0, g)),
        ],
        out_specs=pl.BlockSpec((1, T, VD + MD), lambda b, g: (b, 0, 0)),
        out_shape=jax.ShapeDtypeStruct((B, T, VD + MD), jnp.bfloat16),
    )(vra, mra, vrb, mrb)

    RB = 2
    q, k, v = pl.pallas_call(
        functools.partial(_qkv_kernel, VD),
        grid=(B, RB),
        in_specs=[
            pl.BlockSpec((1, T // RB, VD + MD), lambda b, r: (b, r, 0)),
            pl.BlockSpec((VD + MD, ID), lambda b, r: (0, 0)),
            pl.BlockSpec((VD + MD, ID), lambda b, r: (0, 0)),
            pl.BlockSpec((VD, ID), lambda b, r: (0, 0)),
        ],
        out_specs=[pl.BlockSpec((1, T // RB, ID), lambda b, r: (b, r, 0))] * 3,
        out_shape=[jax.ShapeDtypeStruct((B, T, ID), jnp.bfloat16)] * 3,
    )(x, Wq.astype(jnp.bfloat16), Wk.astype(jnp.bfloat16),
      Wv.astype(jnp.bfloat16))

    HPB = 2
    NH = H // HPB
    hspec = pl.BlockSpec((1, T, HPB * dh), lambda b, h: (b, 0, h))
    u = pl.pallas_call(
        functools.partial(_attn_kernel, 1.0 / np.sqrt(dh), dh, NH),
        grid=(B, NH),
        in_specs=[
            hspec, hspec, hspec,
            pl.BlockSpec((HPB * dh, VD), lambda b, h: (h, 0)),
        ],
        out_specs=pl.BlockSpec((1, T, VD), lambda b, h: (b, 0, 0)),
        out_shape=jax.ShapeDtypeStruct((B, T, VD), jnp.bfloat16),
        scratch_shapes=[pltpu.VMEM((T, VD), jnp.float32)],
    )(q, k, v, Wo.astype(jnp.bfloat16))

    ur = u.reshape(B, 2, K, VD)

    RBROWS = 128
    tab = []
    for s0, s1, d in segs:
        for rb in range(s0 // RBROWS, (s1 - 1) // RBROWS + 1):
            lo = max(s0, rb * RBROWS) - rb * RBROWS
            hi = min(s1, (rb + 1) * RBROWS) - rb * RBROWS
            tab.append((rb, d, lo, hi))
    tab_r = jnp.asarray([e[0] for e in tab], jnp.int32)
    tab_g = jnp.asarray([e[1] for e in tab], jnp.int32)

    ospec = pl.BlockSpec((1, RBROWS, VD),
                         lambda b, t, tr, tg: (b, tr[t], tg[t]))
    oa, ob = pl.pallas_call(
        functools.partial(_add_kernel, tab),
        grid_spec=pltpu.PrefetchScalarGridSpec(
            num_scalar_prefetch=2,
            grid=(B, len(tab)),
            in_specs=[
                ospec, ospec,
                pl.BlockSpec((1, 2, RBROWS, VD),
                             lambda b, t, tr, tg: (b, 0, tr[t], 0)),
            ],
            out_specs=[ospec, ospec],
        ),
        out_shape=[jax.ShapeDtypeStruct((B, R, stride * VD), jnp.float32)] * 2,
        input_output_aliases={2: 0, 3: 1},
    )(tab_r, tab_g, o0a, o0b, ur)
    return oa.reshape(B, N, VD), ob.reshape(B, N, VD)


# speedup vs baseline: 4.2282x; 1.3751x over previous
import functools

import jax
import jax.numpy as jnp
import numpy as np
from jax.experimental import pallas as pl
from jax.experimental.pallas import tpu as pltpu


def _segments(n, k):
    stride = n // k
    idx = np.linspace(0, n - 1, k).astype(np.int64)
    d = idx - stride * np.arange(k)
    segs = []
    s0 = 0
    for i in range(1, k + 1):
        if i == k or d[i] != d[s0]:
            segs.append((int(s0), int(i), int(d[s0])))
            s0 = i
    return stride, segs


def _gather_kernel(segs, kk, vra, mra, vrb, mrb, x):
    g = pl.program_id(1)
    for s0, s1, d in segs:
        @pl.when(g == d)
        def _(s0=s0, s1=s1):
            vd = vra.shape[2]
            md = mra.shape[2]
            x[0, s0:s1, 0:vd] = vra[0, s0:s1, :].astype(jnp.bfloat16)
            x[0, s0:s1, vd:vd + md] = mra[0, s0:s1, :].astype(jnp.bfloat16)
            x[0, kk + s0:kk + s1, 0:vd] = vrb[0, s0:s1, :].astype(jnp.bfloat16)
            x[0, kk + s0:kk + s1, vd:vd + md] = (
                mrb[0, s0:s1, :].astype(jnp.bfloat16))


def _qkv_kernel(vd, x, wq, wk, wv, q, k, v):
    xx = x[0]
    q[0] = jnp.dot(xx, wq[...],
                   preferred_element_type=jnp.float32).astype(jnp.bfloat16)
    k[0] = jnp.dot(xx, wk[...],
                   preferred_element_type=jnp.float32).astype(jnp.bfloat16)
    v[0] = jnp.dot(xx[:, :vd], wv[...],
                   preferred_element_type=jnp.float32).astype(jnp.bfloat16)


def _attn_kernel(scale, dh, nh, q, k, v, wo, u, acc):
    h = pl.program_id(1)
    qq, kk, vv, woo = q[0], k[0], v[0], wo[...]
    contrib = None
    for j in range(qq.shape[-1] // dh):
        qh = qq[:, j * dh:(j + 1) * dh]
        kh = kk[:, j * dh:(j + 1) * dh]
        vh = vv[:, j * dh:(j + 1) * dh]
        s = jax.lax.dot_general(qh, kh, (((1,), (1,)), ((), ())),
                                preferred_element_type=jnp.float32) * scale
        qf = qh.astype(jnp.float32)
        kf = kh.astype(jnp.float32)
        nq = jnp.sqrt(jnp.sum(qf * qf, axis=-1, keepdims=True))
        nk = jnp.sqrt(jnp.max(jnp.sum(kf * kf, axis=-1)))
        m = nq * (nk * scale)
        p = jnp.exp(s - m)
        l = jnp.sum(p, axis=-1, keepdims=True)
        o = jnp.dot(p.astype(jnp.bfloat16), vh,
                    preferred_element_type=jnp.float32) / l
        c = jnp.dot(o.astype(jnp.bfloat16),
                    woo[j * dh:(j + 1) * dh, :],
                    preferred_element_type=jnp.float32)
        contrib = c if contrib is None else contrib + c

    @pl.when(h == 0)
    def _():
        acc[...] = contrib

    @pl.when(h > 0)
    def _():
        acc[...] += contrib

    @pl.when(h == nh - 1)
    def _():
        u[0] = acc[...].astype(jnp.bfloat16)


def _scatter_kernel(segs, vra, vrb, uu, oa, ob):
    g = pl.program_id(1)
    oa[0] = vra[0]
    ob[0] = vrb[0]
    for s0, s1, d in segs:
        @pl.when(g == d)
        def _(s0=s0, s1=s1):
            oa[0, s0:s1, :] += uu[0, 0, s0:s1, :].astype(jnp.float32)
            ob[0, s0:s1, :] += uu[0, 1, s0:s1, :].astype(jnp.float32)


def kernel(values_a, metadata_a, values_b, metadata_b, Wq, Wk, Wv, Wo):
    B, N, VD = values_a.shape
    MD = metadata_a.shape[2]
    ID = Wq.shape[1]
    K = ID
    H = 16
    dh = ID // H

    stride, segs = _segments(N, K)
    R = N // stride

    vra = values_a.reshape(B, R, stride * VD)
    vrb = values_b.reshape(B, R, stride * VD)
    mra = metadata_a.reshape(B, R, stride * MD)
    mrb = metadata_b.reshape(B, R, stride * MD)

    T = 2 * K
    x = pl.pallas_call(
        functools.partial(_gather_kernel, segs, K),
        grid=(B, stride),
        in_specs=[
            pl.BlockSpec((1, R, VD), lambda b, g: (b, 0, g)),
            pl.BlockSpec((1, R, MD), lambda b, g: (b, 0, g)),
            pl.BlockSpec((1, R, VD), lambda b, g: (b, 0, g)),
            pl.BlockSpec((1, R, MD), lambda b, g: (b, 0, g)),
        ],
        out_specs=pl.BlockSpec((1, T, VD + MD), lambda b, g: (b, 0, 0)),
        out_shape=jax.ShapeDtypeStruct((B, T, VD + MD), jnp.bfloat16),
    )(vra, mra, vrb, mrb)

    RB = 2
    q, k, v = pl.pallas_call(
        functools.partial(_qkv_kernel, VD),
        grid=(B, RB),
        in_specs=[
            pl.BlockSpec((1, T // RB, VD + MD), lambda b, r: (b, r, 0)),
            pl.BlockSpec((VD + MD, ID), lambda b, r: (0, 0)),
            pl.BlockSpec((VD + MD, ID), lambda b, r: (0, 0)),
            pl.BlockSpec((VD, ID), lambda b, r: (0, 0)),
        ],
        out_specs=[pl.BlockSpec((1, T // RB, ID), lambda b, r: (b, r, 0))] * 3,
        out_shape=[jax.ShapeDtypeStruct((B, T, ID), jnp.bfloat16)] * 3,
    )(x, Wq.astype(jnp.bfloat16), Wk.astype(jnp.bfloat16),
      Wv.astype(jnp.bfloat16))

    HPB = 4
    NH = H // HPB
    hspec = pl.BlockSpec((1, T, HPB * dh), lambda b, h: (b, 0, h))
    u = pl.pallas_call(
        functools.partial(_attn_kernel, 1.0 / np.sqrt(dh), dh, NH),
        grid=(B, NH),
        in_specs=[
            hspec, hspec, hspec,
            pl.BlockSpec((HPB * dh, VD), lambda b, h: (h, 0)),
        ],
        out_specs=pl.BlockSpec((1, T, VD), lambda b, h: (b, 0, 0)),
        out_shape=jax.ShapeDtypeStruct((B, T, VD), jnp.bfloat16),
        scratch_shapes=[pltpu.VMEM((T, VD), jnp.float32)],
    )(q, k, v, Wo.astype(jnp.bfloat16))

    ur = u.reshape(B, 2, K, VD)

    oa, ob = pl.pallas_call(
        functools.partial(_scatter_kernel, segs),
        grid=(B, stride),
        in_specs=[
            pl.BlockSpec((1, R, VD), lambda b, g: (b, 0, g)),
            pl.BlockSpec((1, R, VD), lambda b, g: (b, 0, g)),
            pl.BlockSpec((1, 2, K, VD), lambda b, g: (b, 0, 0, 0)),
        ],
        out_specs=[pl.BlockSpec((1, R, VD), lambda b, g: (b, 0, g))] * 2,
        out_shape=[jax.ShapeDtypeStruct((B, R, stride * VD), jnp.float32)] * 2,
    )(vra, vrb, ur)
    return oa.reshape(B, N, VD), ob.reshape(B, N, VD)


# speedup vs baseline: 4.2795x; 1.0121x over previous
import functools

import jax
import jax.numpy as jnp
import numpy as np
from jax.experimental import pallas as pl
from jax.experimental.pallas import tpu as pltpu


def _segments(n, k):
    stride = n // k
    idx = np.linspace(0, n - 1, k).astype(np.int64)
    d = idx - stride * np.arange(k)
    segs = []
    s0 = 0
    for i in range(1, k + 1):
        if i == k or d[i] != d[s0]:
            segs.append((int(s0), int(i), int(d[s0])))
            s0 = i
    return stride, segs


def _gather_kernel(tab, rbrows, kk, tabr, tabg, vra, mra, vrb, mrb, x):
    t = pl.program_id(1)
    vd = vra.shape[2]
    md = mra.shape[2]
    for tt, (rb, _, lo, hi) in enumerate(tab):
        @pl.when(t == tt)
        def _(rb=rb, lo=lo, hi=hi):
            r0 = rb * rbrows
            x[0, r0 + lo:r0 + hi, 0:vd] = (
                vra[0, lo:hi, :].astype(jnp.bfloat16))
            x[0, r0 + lo:r0 + hi, vd:vd + md] = (
                mra[0, lo:hi, :].astype(jnp.bfloat16))
            x[0, kk + r0 + lo:kk + r0 + hi, 0:vd] = (
                vrb[0, lo:hi, :].astype(jnp.bfloat16))
            x[0, kk + r0 + lo:kk + r0 + hi, vd:vd + md] = (
                mrb[0, lo:hi, :].astype(jnp.bfloat16))


def _qkv_kernel(vd, x, wq, wk, wv, q, k, v):
    xx = x[0]
    q[0] = jnp.dot(xx, wq[...],
                   preferred_element_type=jnp.float32).astype(jnp.bfloat16)
    k[0] = jnp.dot(xx, wk[...],
                   preferred_element_type=jnp.float32).astype(jnp.bfloat16)
    v[0] = jnp.dot(xx[:, :vd], wv[...],
                   preferred_element_type=jnp.float32).astype(jnp.bfloat16)


def _attn_kernel(scale, dh, nh, q, k, v, wo, u, acc):
    h = pl.program_id(1)
    qq, kk, vv, woo = q[0], k[0], v[0], wo[...]
    contrib = None
    for j in range(qq.shape[-1] // dh):
        qh = qq[:, j * dh:(j + 1) * dh]
        kh = kk[:, j * dh:(j + 1) * dh]
        vh = vv[:, j * dh:(j + 1) * dh]
        s = jax.lax.dot_general(qh, kh, (((1,), (1,)), ((), ())),
                                preferred_element_type=jnp.float32) * scale
        qf = qh.astype(jnp.float32)
        kf = kh.astype(jnp.float32)
        nq = jnp.sqrt(jnp.sum(qf * qf, axis=-1, keepdims=True))
        nk = jnp.sqrt(jnp.max(jnp.sum(kf * kf, axis=-1)))
        m = nq * (nk * scale)
        p = jnp.exp(s - m)
        l = jnp.sum(p, axis=-1, keepdims=True)
        o = jnp.dot(p.astype(jnp.bfloat16), vh,
                    preferred_element_type=jnp.float32) / l
        c = jnp.dot(o.astype(jnp.bfloat16),
                    woo[j * dh:(j + 1) * dh, :],
                    preferred_element_type=jnp.float32)
        contrib = c if contrib is None else contrib + c

    @pl.when(h == 0)
    def _():
        acc[...] = contrib

    @pl.when(h > 0)
    def _():
        acc[...] += contrib

    @pl.when(h == nh - 1)
    def _():
        u[0] = acc[...].astype(jnp.bfloat16)


def _scatter_kernel(segs, vra, vrb, uu, oa, ob):
    g = pl.program_id(1)
    oa[0] = vra[0]
    ob[0] = vrb[0]
    for s0, s1, d in segs:
        @pl.when(g == d)
        def _(s0=s0, s1=s1):
            oa[0, s0:s1, :] += uu[0, 0, s0:s1, :].astype(jnp.float32)
            ob[0, s0:s1, :] += uu[0, 1, s0:s1, :].astype(jnp.float32)


def kernel(values_a, metadata_a, values_b, metadata_b, Wq, Wk, Wv, Wo):
    B, N, VD = values_a.shape
    MD = metadata_a.shape[2]
    ID = Wq.shape[1]
    K = ID
    H = 16
    dh = ID // H

    stride, segs = _segments(N, K)
    R = N // stride

    vra = values_a.reshape(B, R, stride * VD)
    vrb = values_b.reshape(B, R, stride * VD)
    mra = metadata_a.reshape(B, R, stride * MD)
    mrb = metadata_b.reshape(B, R, stride * MD)

    T = 2 * K
    RBROWS = 128
    tab = []
    for s0, s1, d in segs:
        for rb in range(s0 // RBROWS, (s1 - 1) // RBROWS + 1):
            lo = max(s0, rb * RBROWS) - rb * RBROWS
            hi = min(s1, (rb + 1) * RBROWS) - rb * RBROWS
            tab.append((rb, d, lo, hi))
    tab_r = jnp.asarray([e[0] for e in tab], jnp.int32)
    tab_g = jnp.asarray([e[1] for e in tab], jnp.int32)

    x = pl.pallas_call(
        functools.partial(_gather_kernel, tab, RBROWS, K),
        grid_spec=pltpu.PrefetchScalarGridSpec(
            num_scalar_prefetch=2,
            grid=(B, len(tab)),
            in_specs=[
                pl.BlockSpec((1, RBROWS, VD),
                             lambda b, t, tr, tg: (b, tr[t], tg[t])),
                pl.BlockSpec((1, RBROWS, MD),
                             lambda b, t, tr, tg: (b, tr[t], tg[t])),
                pl.BlockSpec((1, RBROWS, VD),
                             lambda b, t, tr, tg: (b, tr[t], tg[t])),
                pl.BlockSpec((1, RBROWS, MD),
                             lambda b, t, tr, tg: (b, tr[t], tg[t])),
            ],
            out_specs=pl.BlockSpec((1, T, VD + MD),
                                   lambda b, t, tr, tg: (b, 0, 0)),
        ),
        out_shape=jax.ShapeDtypeStruct((B, T, VD + MD), jnp.bfloat16),
    )(tab_r, tab_g, vra, mra, vrb, mrb)

    RB = 2
    q, k, v = pl.pallas_call(
        functools.partial(_qkv_kernel, VD),
        grid=(B, RB),
        in_specs=[
            pl.BlockSpec((1, T // RB, VD + MD), lambda b, r: (b, r, 0)),
            pl.BlockSpec((VD + MD, ID), lambda b, r: (0, 0)),
            pl.BlockSpec((VD + MD, ID), lambda b, r: (0, 0)),
            pl.BlockSpec((VD, ID), lambda b, r: (0, 0)),
        ],
        out_specs=[pl.BlockSpec((1, T // RB, ID), lambda b, r: (b, r, 0))] * 3,
        out_shape=[jax.ShapeDtypeStruct((B, T, ID), jnp.bfloat16)] * 3,
    )(x, Wq.astype(jnp.bfloat16), Wk.astype(jnp.bfloat16),
      Wv.astype(jnp.bfloat16))

    HPB = 4
    NH = H // HPB
    hspec = pl.BlockSpec((1, T, HPB * dh), lambda b, h: (b, 0, h))
    u = pl.pallas_call(
        functools.partial(_attn_kernel, 1.0 / np.sqrt(dh), dh, NH),
        grid=(B, NH),
        in_specs=[
            hspec, hspec, hspec,
            pl.BlockSpec((HPB * dh, VD), lambda b, h: (h, 0)),
        ],
        out_specs=pl.BlockSpec((1, T, VD), lambda b, h: (b, 0, 0)),
        out_shape=jax.ShapeDtypeStruct((B, T, VD), jnp.bfloat16),
        scratch_shapes=[pltpu.VMEM((T, VD), jnp.float32)],
    )(q, k, v, Wo.astype(jnp.bfloat16))

    ur = u.reshape(B, 2, K, VD)

    oa, ob = pl.pallas_call(
        functools.partial(_scatter_kernel, segs),
        grid=(B, stride),
        in_specs=[
            pl.BlockSpec((1, R, VD), lambda b, g: (b, 0, g)),
            pl.BlockSpec((1, R, VD), lambda b, g: (b, 0, g)),
            pl.BlockSpec((1, 2, K, VD), lambda b, g: (b, 0, 0, 0)),
        ],
        out_specs=[pl.BlockSpec((1, R, VD), lambda b, g: (b, 0, g))] * 2,
        out_shape=[jax.ShapeDtypeStruct((B, R, stride * VD), jnp.float32)] * 2,
    )(vra, vrb, ur)
    return oa.reshape(B, N, VD), ob.reshape(B, N, VD)
